# baseline (speedup 1.0000x reference)
import functools

import jax
import jax.numpy as jnp
from jax.experimental import pallas as pl
from jax.experimental.pallas import tpu as pltpu

_MIB = 1024 * 1024


def _fused_kernel(x_ref, w_ref, a_ref, b_ref, o_ref, acc_ref):
    c = pl.program_id(1)
    last = pl.num_programs(1) - 1

    s = jnp.sum(x_ref[0], axis=(1, 2))[:, None]
    part = jax.lax.dot_general(w_ref[...], s,
                               (((1,), (0,)), ((), ())),
                               preferred_element_type=jnp.float32)

    @pl.when(c == 0)
    def _():
        acc_ref[...] = part

    @pl.when(c > 0)
    def _():
        acc_ref[...] += part

    @pl.when(c == last)
    def _():
        z = jnp.maximum(acc_ref[...] * a_ref[...] + b_ref[...], 0.0)
        o_ref[0] = jnp.broadcast_to(z[:, :, None],
                                    o_ref.shape[1:]).astype(o_ref.dtype)


def kernel(x, conv_w, bn_gamma, bn_beta, bn_mean, bn_var, eps=1e-5):
    N, Cin, H, W = x.shape
    Cout = conv_w.shape[0]
    HW = H * W

    scale = (bn_gamma.astype(jnp.float32)
             / jnp.sqrt(bn_var.astype(jnp.float32) + eps))
    bias = bn_beta.astype(jnp.float32) - bn_mean.astype(jnp.float32) * scale
    alpha = (scale * (1.0 / HW))[:, None]
    beta = bias[:, None]
    wr = conv_w.reshape(Cout, Cin).astype(jnp.float32)

    cin_tile = 256 if Cin % 256 == 0 else Cin
    n_cin = Cin // cin_tile
    itemsize = jnp.dtype(x.dtype).itemsize

    out = pl.pallas_call(
        _fused_kernel,
        out_shape=jax.ShapeDtypeStruct((N, Cout, H, W), x.dtype),
        grid=(N, n_cin),
        in_specs=[
            pl.BlockSpec((1, cin_tile, H, W), lambda n, c: (n, c, 0, 0)),
            pl.BlockSpec((Cout, cin_tile), lambda n, c: (0, c)),
            pl.BlockSpec((Cout, 1), lambda n, c: (0, 0)),
            pl.BlockSpec((Cout, 1), lambda n, c: (0, 0)),
        ],
        out_specs=pl.BlockSpec((1, Cout, H, W), lambda n, c: (n, 0, 0, 0)),
        scratch_shapes=[pltpu.VMEM((Cout, 1), jnp.float32)],
        compiler_params=pltpu.CompilerParams(
            dimension_semantics=("parallel", "arbitrary"),
            vmem_limit_bytes=48 * _MIB),
        cost_estimate=pl.CostEstimate(
            flops=int(N * Cin * HW + 2 * N * Cin * Cout),
            transcendentals=0,
            bytes_accessed=int(N * Cin * HW * itemsize
                               + N * Cout * HW * itemsize
                               + Cin * Cout * 4)),
    )(x, wr, alpha, beta)

    return out


# speedup vs baseline: 12.2341x; 12.2341x over previous
import jax
import jax.numpy as jnp
from jax.experimental import pallas as pl
from jax.experimental.pallas import tpu as pltpu

_MIB = 1024 * 1024


def _fused_kernel(x_ref, w_ref, a_ref, b_ref, o_ref):
    s = jnp.sum(x_ref[0], axis=0, keepdims=True)
    y = jax.lax.dot_general(s, w_ref[...],
                            (((1,), (1,)), ((), ())),
                            preferred_element_type=jnp.float32)
    z = jnp.maximum(y * a_ref[...] + b_ref[...], 0.0)
    o_ref[0] = jnp.broadcast_to(z, o_ref.shape[1:]).astype(o_ref.dtype)


def kernel(x, conv_w, bn_gamma, bn_beta, bn_mean, bn_var, eps=1e-5):
    N, Cin, H, W = x.shape
    Cout = conv_w.shape[0]
    HW = H * W

    scale = (bn_gamma.astype(jnp.float32)
             / jnp.sqrt(bn_var.astype(jnp.float32) + eps))
    bias = bn_beta.astype(jnp.float32) - bn_mean.astype(jnp.float32) * scale
    alpha = (scale * (1.0 / HW))[None, :]
    beta = bias[None, :]
    wr = conv_w.reshape(Cout, Cin).astype(jnp.float32)

    xv = jnp.transpose(x, (0, 2, 3, 1)).reshape(N, HW, Cin)
    itemsize = jnp.dtype(x.dtype).itemsize

    out = pl.pallas_call(
        _fused_kernel,
        out_shape=jax.ShapeDtypeStruct((N, HW, Cout), x.dtype),
        grid=(N,),
        in_specs=[
            pl.BlockSpec((1, HW, Cin), lambda n: (n, 0, 0)),
            pl.BlockSpec((Cout, Cin), lambda n: (0, 0)),
            pl.BlockSpec((1, Cout), lambda n: (0, 0)),
            pl.BlockSpec((1, Cout), lambda n: (0, 0)),
        ],
        out_specs=pl.BlockSpec((1, HW, Cout), lambda n: (n, 0, 0)),
        compiler_params=pltpu.CompilerParams(
            dimension_semantics=("parallel",),
            vmem_limit_bytes=48 * _MIB),
        cost_estimate=pl.CostEstimate(
            flops=int(N * Cin * HW + 2 * N * Cin * Cout),
            transcendentals=0,
            bytes_accessed=int(N * Cin * HW * itemsize
                               + N * Cout * HW * itemsize
                               + Cin * Cout * 4)),
    )(xv, wr, alpha, beta)

    return out.reshape(N, H, W, Cout).transpose(0, 3, 1, 2)


# speedup vs baseline: 12.5076x; 1.0224x over previous
import jax
import jax.numpy as jnp
from jax.experimental import pallas as pl
from jax.experimental.pallas import tpu as pltpu

_MIB = 1024 * 1024


def _fused_kernel(x_ref, w_ref, a_ref, b_ref, o_ref):
    cout, k, _ = w_ref.shape
    s = jnp.sum(x_ref[0], axis=0, keepdims=True)
    w2d = w_ref[...].reshape(cout, k * 128)
    y = jax.lax.dot_general(s, w2d,
                            (((1,), (1,)), ((), ())),
                            preferred_element_type=jnp.float32)
    z = jnp.maximum(y * a_ref[...] + b_ref[...], 0.0)
    o_ref[0] = jnp.broadcast_to(z, o_ref.shape[1:]).astype(o_ref.dtype)


def kernel(x, conv_w, bn_gamma, bn_beta, bn_mean, bn_var, eps=1e-5):
    N, Cin, H, W = x.shape
    Cout = conv_w.shape[0]
    HW = H * W

    scale = (bn_gamma.astype(jnp.float32)
             / jnp.sqrt(bn_var.astype(jnp.float32) + eps))
    bias = bn_beta.astype(jnp.float32) - bn_mean.astype(jnp.float32) * scale
    alpha = (scale * (1.0 / HW))[None, :]
    beta = bias[None, :]
    wr = conv_w.reshape(Cout, Cin // 128, 128).astype(jnp.float32)

    xv = jnp.transpose(x, (0, 2, 3, 1)).reshape(N, HW, Cin)
    itemsize = jnp.dtype(x.dtype).itemsize

    out = pl.pallas_call(
        _fused_kernel,
        out_shape=jax.ShapeDtypeStruct((N, HW, Cout), x.dtype),
        grid=(N,),
        in_specs=[
            pl.BlockSpec((1, HW, Cin), lambda n: (n, 0, 0)),
            pl.BlockSpec((Cout, Cin // 128, 128), lambda n: (0, 0, 0)),
            pl.BlockSpec((1, Cout), lambda n: (0, 0)),
            pl.BlockSpec((1, Cout), lambda n: (0, 0)),
        ],
        out_specs=pl.BlockSpec((1, HW, Cout), lambda n: (n, 0, 0)),
        compiler_params=pltpu.CompilerParams(
            dimension_semantics=("parallel",),
            vmem_limit_bytes=48 * _MIB),
        cost_estimate=pl.CostEstimate(
            flops=int(N * Cin * HW + 2 * N * Cin * Cout),
            transcendentals=0,
            bytes_accessed=int(N * Cin * HW * itemsize
                               + N * Cout * HW * itemsize
                               + Cin * Cout * 4)),
    )(xv, wr, alpha, beta)

    return out.reshape(N, H, W, Cout).transpose(0, 3, 1, 2)
